# baseline (speedup 1.0000x reference)
import functools

import jax
import jax.numpy as jnp
from jax import lax
from jax.experimental import pallas as pl
from jax.experimental.pallas import tpu as pltpu
from jax.experimental.pallas import tpu_sc as plsc

VOCAB = 1000000
D = 64
DPAD = 128
BATCH = 4096
HIST = 200
B_FLAT = BATCH * HIST

NW = 32
CHUNK = 200
NSLOT = 4
N_PW = B_FLAT // NW
N_CHUNKS = N_PW // CHUNK


def _emb_kernel(idx_hbm, table_hbm, out_hbm, idx_v, rows_v, gsem, osem):
    wid = lax.axis_index("s") * 2 + lax.axis_index("c")
    base = wid * N_PW

    pltpu.sync_copy(idx_hbm.at[pl.ds(base, N_PW)], idx_v)

    def fire_gather(i, slot):
        pltpu.async_copy(
            table_hbm.at[idx_v.at[pl.ds(i * CHUNK, CHUNK)]],
            rows_v.at[slot],
            gsem,
        )

    def drain_gather():
        pltpu.make_async_copy(
            out_hbm.at[pl.ds(0, CHUNK)], rows_v.at[0], gsem
        ).wait()

    def fire_write(i, slot):
        pltpu.async_copy(
            rows_v.at[slot], out_hbm.at[pl.ds(base + i * CHUNK, CHUNK)], osem
        )

    def drain_write():
        pltpu.make_async_copy(
            rows_v.at[0], out_hbm.at[pl.ds(0, CHUNK)], osem
        ).wait()

    fire_gather(0, 0)
    fire_gather(1, 1)
    fire_gather(2, 2)

    def body(i, carry):
        slot = lax.rem(i, NSLOT)
        nslot = lax.rem(i + 3, NSLOT)

        @pl.when(i >= 1)
        def _():
            drain_write()

        @pl.when(i + 3 < N_CHUNKS)
        def _():
            fire_gather(i + 3, nslot)

        drain_gather()
        fire_write(i, slot)
        return carry

    lax.fori_loop(0, N_CHUNKS, body, 0)
    drain_write()


@functools.partial(
    pl.kernel,
    out_type=jax.ShapeDtypeStruct((B_FLAT, DPAD), jnp.float32),
    scratch_types=[
        pltpu.VMEM((N_PW,), jnp.int32),
        pltpu.VMEM((NSLOT, CHUNK, DPAD), jnp.float32),
        pltpu.SemaphoreType.DMA,
        pltpu.SemaphoreType.DMA,
    ],
    mesh=plsc.VectorSubcoreMesh(core_axis_name="c", subcore_axis_name="s"),
    compiler_params=pltpu.CompilerParams(use_tc_tiling_on_sc=True),
)
def _emb(idx_hbm, table_hbm, out_hbm, idx_v, rows_v, gsem, osem):
    _emb_kernel(idx_hbm, table_hbm, out_hbm, idx_v, rows_v, gsem, osem)


def kernel(x, weights):
    proj = jnp.concatenate(
        [jnp.eye(D, dtype=jnp.float32),
         jnp.zeros((D, DPAD - D), jnp.float32)], axis=1)
    wpad = jax.lax.dot(weights, proj)
    xf = x.astype(jnp.int32).reshape(B_FLAT)
    outp = _emb(xf, wpad)
    return outp.reshape(BATCH, HIST, DPAD)[..., :D]


# speedup vs baseline: 1.0032x; 1.0032x over previous
import functools

import jax
import jax.numpy as jnp
from jax import lax
from jax.experimental import pallas as pl
from jax.experimental.pallas import tpu as pltpu
from jax.experimental.pallas import tpu_sc as plsc

VOCAB = 1000000
D = 64
DPAD = 128
BATCH = 4096
HIST = 200
B_FLAT = BATCH * HIST

NW = 32
CHUNK = 256
NSLOT = 3
N_PW = B_FLAT // NW
N_CHUNKS = N_PW // CHUNK


def _emb_kernel(idx_hbm, table_hbm, out_hbm, idx_v, rows_v, gsem, osem):
    wid = lax.axis_index("s") * 2 + lax.axis_index("c")
    base = wid * N_PW

    pltpu.sync_copy(idx_hbm.at[pl.ds(base, N_PW)], idx_v)

    H = CHUNK // 2

    def fire_gather(i, slot):
        pltpu.async_copy(
            table_hbm.at[idx_v.at[pl.ds(i * CHUNK, H)]],
            rows_v.at[slot, pl.ds(0, H)],
            gsem,
        )
        pltpu.async_copy(
            table_hbm.at[idx_v.at[pl.ds(i * CHUNK + H, H)]],
            rows_v.at[slot, pl.ds(H, H)],
            gsem,
        )

    def drain_gather():
        pltpu.make_async_copy(
            out_hbm.at[pl.ds(0, H)], rows_v.at[0, pl.ds(0, H)], gsem
        ).wait()
        pltpu.make_async_copy(
            out_hbm.at[pl.ds(0, H)], rows_v.at[0, pl.ds(0, H)], gsem
        ).wait()

    def fire_write(i, slot):
        pltpu.async_copy(
            rows_v.at[slot], out_hbm.at[pl.ds(base + i * CHUNK, CHUNK)], osem
        )

    def drain_write():
        pltpu.make_async_copy(
            rows_v.at[0], out_hbm.at[pl.ds(0, CHUNK)], osem
        ).wait()

    fire_gather(0, 0)
    fire_gather(1, 1)

    def body(i, carry):
        slot = lax.rem(i, NSLOT)
        nslot = lax.rem(i + 2, NSLOT)

        @pl.when(i >= 1)
        def _():
            drain_write()

        @pl.when(i + 2 < N_CHUNKS)
        def _():
            fire_gather(i + 2, nslot)

        drain_gather()
        fire_write(i, slot)
        return carry

    lax.fori_loop(0, N_CHUNKS, body, 0)
    drain_write()


@functools.partial(
    pl.kernel,
    out_type=jax.ShapeDtypeStruct((B_FLAT, DPAD), jnp.float32),
    scratch_types=[
        pltpu.VMEM((N_PW,), jnp.int32),
        pltpu.VMEM((NSLOT, CHUNK, DPAD), jnp.float32),
        pltpu.SemaphoreType.DMA,
        pltpu.SemaphoreType.DMA,
    ],
    mesh=plsc.VectorSubcoreMesh(core_axis_name="c", subcore_axis_name="s"),
    compiler_params=pltpu.CompilerParams(use_tc_tiling_on_sc=True),
)
def _emb(idx_hbm, table_hbm, out_hbm, idx_v, rows_v, gsem, osem):
    _emb_kernel(idx_hbm, table_hbm, out_hbm, idx_v, rows_v, gsem, osem)


def kernel(x, weights):
    proj = jnp.concatenate(
        [jnp.eye(D, dtype=jnp.float32),
         jnp.zeros((D, DPAD - D), jnp.float32)], axis=1)
    wpad = jax.lax.dot(weights, proj)
    xf = x.astype(jnp.int32).reshape(B_FLAT)
    outp = _emb(xf, wpad)
    return outp.reshape(BATCH, HIST, DPAD)[..., :D]
